# baseline (speedup 1.0000x reference)
import functools

import jax
import jax.numpy as jnp
from jax import lax
from jax.experimental import pallas as pl
from jax.experimental.pallas import tpu as pltpu
from jax.experimental.pallas import tpu_sc as plsc

_NUM_CLASSES = 1000
_MULT = 26
_ROWS = 1024
_LANES = 16


def _make_sc_kernel():
    info = plsc.get_sparse_core_info()
    nc, ns = info.num_cores, info.num_subcores
    nw = nc * ns
    rows_per_w = _ROWS // nw
    mesh = plsc.VectorSubcoreMesh(core_axis_name="c", subcore_axis_name="s")

    @functools.partial(
        pl.kernel,
        out_type=jax.ShapeDtypeStruct((_ROWS, _MULT, _NUM_CLASSES), jnp.int32),
        mesh=mesh,
        scratch_types=[
            pltpu.VMEM((rows_per_w, _MULT), jnp.int32),
            pltpu.VMEM((_MULT, _NUM_CLASSES), jnp.int32),
            pltpu.VMEM((_MULT, _NUM_CLASSES), jnp.int32),
            pltpu.SemaphoreType.DMA,
            pltpu.SemaphoreType.DMA,
        ],
        compiler_params=pltpu.CompilerParams(
            use_tc_tiling_on_sc=False, needs_layout_passes=False
        ),
    )
    def k(feat_hbm, out_hbm, feat_v, buf0, buf1, sem0, sem1):
        wid = lax.axis_index("s") * nc + lax.axis_index("c")
        base = wid * rows_per_w
        pltpu.sync_copy(feat_hbm.at[pl.ds(base, rows_per_w)], feat_v)

        iota = lax.iota(jnp.int32, _LANES)
        ones = jnp.full((_LANES,), 1, jnp.int32)
        zeros = jnp.zeros((_LANES,), jnp.int32)
        bufs = (buf0, buf1)
        sems = (sem0, sem1)

        def zero_row(j, carry):
            for b in range(2):
                for c in range(62):
                    bufs[b][j, pl.ds(c * _LANES, _LANES)] = zeros
                bufs[b][j, pl.ds(_NUM_CLASSES - _LANES, _LANES)] = zeros
            return carry

        lax.fori_loop(0, _MULT, zero_row, 0)

        def scatter_row(buf, r, val):
            f0 = feat_v[r, pl.ds(0, _LANES)]
            f1 = feat_v[r, pl.ds(_MULT - _LANES, _LANES)]
            plsc.store_scatter(buf, [iota, f0], val)
            plsc.store_scatter(buf, [iota + (_MULT - _LANES), f1], val)

        pend = [None, None]
        for r in range(rows_per_w):
            b = r & 1
            if pend[b] is not None:
                pend[b].wait()
                scatter_row(bufs[b], r - 2, zeros)
            scatter_row(bufs[b], r, ones)
            pend[b] = pltpu.async_copy(bufs[b], out_hbm.at[base + r], sems[b])
        for b in range(2):
            if pend[b] is not None:
                pend[b].wait()

    return k


_sc_onehot = _make_sc_kernel()


def kernel(feat):
    return _sc_onehot(feat)


# speedup vs baseline: 2.0416x; 2.0416x over previous
import jax
import jax.numpy as jnp
from jax.experimental import pallas as pl
from jax.experimental.pallas import tpu as pltpu

_NUM_CLASSES = 1000
_MULT = 26
_ROWS = 1024
_BLOCK_ROWS = 32
_NBUF = 8
_STEPS = _ROWS // _BLOCK_ROWS


def _onehot_body(feat_ref, out_hbm, scratch, sems):
    i = pl.program_id(0)
    b = jax.lax.rem(i, _NBUF)

    @pl.when(i >= _NBUF)
    def _wait_oldest():
        rows = (i - _NBUF) * _BLOCK_ROWS
        pltpu.make_async_copy(
            scratch.at[b], out_hbm.at[pl.ds(rows, _BLOCK_ROWS)], sems.at[b]
        ).wait()

    f = feat_ref[...]
    classes = jax.lax.broadcasted_iota(
        jnp.int32, (_BLOCK_ROWS, _MULT, _NUM_CLASSES), 2
    )
    scratch[b] = (f[:, :, None] == classes).astype(jnp.int32)

    pltpu.make_async_copy(
        scratch.at[b],
        out_hbm.at[pl.ds(i * _BLOCK_ROWS, _BLOCK_ROWS)],
        sems.at[b],
    ).start()

    @pl.when(i == _STEPS - 1)
    def _drain():
        for k in range(_NBUF):
            step = _STEPS - _NBUF + k
            kb = step % _NBUF
            pltpu.make_async_copy(
                scratch.at[kb],
                out_hbm.at[pl.ds(step * _BLOCK_ROWS, _BLOCK_ROWS)],
                sems.at[kb],
            ).wait()


def kernel(feat):
    return pl.pallas_call(
        _onehot_body,
        grid=(_STEPS,),
        in_specs=[pl.BlockSpec((_BLOCK_ROWS, _MULT), lambda i: (i, 0))],
        out_specs=pl.BlockSpec(memory_space=pl.ANY),
        out_shape=jax.ShapeDtypeStruct((_ROWS, _MULT, _NUM_CLASSES), jnp.int32),
        scratch_shapes=[
            pltpu.VMEM((_NBUF, _BLOCK_ROWS, _MULT, _NUM_CLASSES), jnp.int32),
            pltpu.SemaphoreType.DMA((_NBUF,)),
        ],
    )(feat)


# speedup vs baseline: 9.6085x; 4.7065x over previous
import jax
import jax.numpy as jnp
from jax.experimental import pallas as pl

_NUM_CLASSES = 1000
_MULT = 26
_ROWS = 1024
_BLOCK_BATCH = 128


def _onehot_block(featT_ref, out_ref):
    f = featT_ref[...]
    classes = jax.lax.broadcasted_iota(
        jnp.int32, (_MULT, _NUM_CLASSES, _BLOCK_BATCH), 1
    )
    out_ref[...] = (f[:, None, :] == classes).astype(jnp.int32)


def kernel(feat):
    featT = feat.T
    grid = (_ROWS // _BLOCK_BATCH,)
    t = pl.pallas_call(
        _onehot_block,
        grid=grid,
        in_specs=[pl.BlockSpec((_MULT, _BLOCK_BATCH), lambda i: (0, i))],
        out_specs=pl.BlockSpec(
            (_MULT, _NUM_CLASSES, _BLOCK_BATCH), lambda i: (0, 0, i)
        ),
        out_shape=jax.ShapeDtypeStruct((_MULT, _NUM_CLASSES, _ROWS), jnp.int32),
    )(featT)
    return jnp.transpose(t, (2, 0, 1))
